# baseline (speedup 1.0000x reference)
import jax
import jax.numpy as jnp
from jax import lax
from jax.experimental import pallas as pl
from jax.experimental.pallas import tpu as pltpu
from jax.experimental.pallas import tpu_sc as plsc

NUM_CORES = 2
NUM_SUBCORES = 16
NW = NUM_CORES * NUM_SUBCORES
LANES = 16
DIM = 64
PACK = 2 * DIM
TOTAL = 2 * 16384
B_PER_W = TOTAL // NW
CHUNK = 128
NCHUNK = B_PER_W // CHUNK
SUBBLK = CHUNK // LANES

NUM_ENT = 1000000
EBLK = 1024
TGRID = (NUM_ENT + EBLK - 1) // EBLK


def _tr_body(x_ref, o_ref):
    i0 = lax.broadcasted_iota(jnp.int32, (DIM, DIM), 0)
    i1 = lax.broadcasted_iota(jnp.int32, (DIM, DIM), 1)
    eye = (i0 == i1).astype(jnp.float32)
    xt = lax.dot_general(x_ref[...], eye, (((0,), (0,)), ((), ())),
                         preferred_element_type=jnp.float32)
    o_ref[...] = jnp.concatenate([xt[: EBLK // 2, :], xt[EBLK // 2 :, :]], axis=1)


def _sqrt_f32(x):
    i = plsc.bitcast(x, jnp.int32)
    y = plsc.bitcast(jnp.int32(0x5F3759DF) - (i >> 1), jnp.float32)
    for _ in range(3):
        y = y * (1.5 - 0.5 * x * y * y)
    return x * y


def _transe_body(ent_hbm, rel_hbm, hd_hbm, hp_hbm, rd_hbm, rp_hbm,
                 td_hbm, tp_hbm, out_hbm,
                 hdx, hpx, rdx, rpx, tdx, tpx,
                 hrows0, hrows1, rrows0, rrows1, trows0, trows1,
                 sums, sem):
    wid = lax.axis_index("s") * NUM_CORES + lax.axis_index("c")

    cps = [pltpu.async_copy(src.at[wid], dst, sem)
           for src, dst in ((hd_hbm, hdx), (hp_hbm, hpx), (rd_hbm, rdx),
                            (rp_hbm, rpx), (td_hbm, tdx), (tp_hbm, tpx))]
    for cp in cps:
        cp.wait()

    hbufs = (hrows0, hrows1)
    rbufs = (rrows0, rrows1)
    tbufs = (trows0, trows1)

    def fire(c):
        hb, rb, tb = hbufs[c % 2], rbufs[c % 2], tbufs[c % 2]
        return [
            pltpu.async_copy(ent_hbm.at[hdx.at[c]], hb, sem),
            pltpu.async_copy(rel_hbm.at[rdx.at[c]], rb, sem),
            pltpu.async_copy(ent_hbm.at[tdx.at[c]], tb, sem),
        ]

    lane_iota = lax.iota(jnp.int32, LANES)

    def compute(c):
        hb, rb, tb = hbufs[c % 2], rbufs[c % 2], tbufs[c % 2]

        def blk_body(s, carry):
            off = s * LANES
            rowv = off + lane_iota
            ph = hpx[c, pl.ds(off, LANES)]
            pr = rpx[c, pl.ds(off, LANES)]
            pt = tpx[c, pl.ds(off, LANES)]
            acc = jnp.zeros((LANES,), jnp.float32)
            for j in range(DIM):
                hv = plsc.load_gather(hb, [rowv, ph + j])
                rv = plsc.load_gather(rb, [rowv, pr + j])
                tv = plsc.load_gather(tb, [rowv, pt + j])
                d = hv + rv - tv
                acc = acc + d * d
            sums[pl.ds(c * CHUNK + off, LANES)] = _sqrt_f32(acc)
            return carry

        lax.fori_loop(0, SUBBLK, blk_body, 0)

    inflight = fire(0)
    for c in range(NCHUNK):
        nxt = fire(c + 1) if c + 1 < NCHUNK else []
        for cp in inflight:
            cp.wait()
        compute(c)
        inflight = nxt

    pltpu.sync_copy(sums, out_hbm.at[pl.ds(wid * B_PER_W, B_PER_W)])


@jax.jit
def kernel(entity_embeddings, relation_embeddings,
           positive_head_batch, positive_relation_batch, positive_tail_batch,
           negative_head_batch, negative_relation_batch, negative_tail_batch):
    ent2 = pl.pallas_call(
        _tr_body,
        grid=(TGRID,),
        in_specs=[pl.BlockSpec((DIM, EBLK), lambda i: (0, i))],
        out_specs=pl.BlockSpec((EBLK // 2, PACK), lambda i: (i, 0)),
        out_shape=jax.ShapeDtypeStruct((TGRID * EBLK // 2, PACK), jnp.float32),
    )(entity_embeddings.T)
    rel2 = relation_embeddings.reshape(500, PACK)

    def prep_ent(a, b):
        ids = jnp.concatenate([a, b]).astype(jnp.int32)
        div = (((ids >> 10) << 9) | (ids & 511)).reshape(NW, NCHUNK, CHUNK)
        par = (((ids >> 9) & 1) << 6).reshape(NW, NCHUNK, CHUNK)
        return div, par

    def prep_rel(a, b):
        ids = jnp.concatenate([a, b]).astype(jnp.int32)
        div = (ids >> 1).reshape(NW, NCHUNK, CHUNK)
        par = ((ids & 1) << 6).reshape(NW, NCHUNK, CHUNK)
        return div, par

    hd, hp = prep_ent(positive_head_batch, negative_head_batch)
    rd, rp = prep_rel(positive_relation_batch, negative_relation_batch)
    td, tp = prep_ent(positive_tail_batch, negative_tail_batch)

    k = pl.kernel(
        _transe_body,
        out_type=jax.ShapeDtypeStruct((TOTAL,), jnp.float32),
        mesh=plsc.VectorSubcoreMesh(core_axis_name="c", subcore_axis_name="s"),
        scratch_types=[
            pltpu.VMEM((NCHUNK, CHUNK), jnp.int32),
            pltpu.VMEM((NCHUNK, CHUNK), jnp.int32),
            pltpu.VMEM((NCHUNK, CHUNK), jnp.int32),
            pltpu.VMEM((NCHUNK, CHUNK), jnp.int32),
            pltpu.VMEM((NCHUNK, CHUNK), jnp.int32),
            pltpu.VMEM((NCHUNK, CHUNK), jnp.int32),
            pltpu.VMEM((CHUNK, PACK), jnp.float32),
            pltpu.VMEM((CHUNK, PACK), jnp.float32),
            pltpu.VMEM((CHUNK, PACK), jnp.float32),
            pltpu.VMEM((CHUNK, PACK), jnp.float32),
            pltpu.VMEM((CHUNK, PACK), jnp.float32),
            pltpu.VMEM((CHUNK, PACK), jnp.float32),
            pltpu.VMEM((B_PER_W,), jnp.float32),
            pltpu.SemaphoreType.DMA,
        ],
        compiler_params=pltpu.CompilerParams(
            needs_layout_passes=False, use_tc_tiling_on_sc=True),
        name="transe_sc",
    )
    losses = k(ent2, rel2, hd, hp, rd, rp, td, tp)
    return losses.reshape(2, 16384)


# speedup vs baseline: 1.1616x; 1.1616x over previous
import jax
import jax.numpy as jnp
from jax import lax
from jax.experimental import pallas as pl
from jax.experimental.pallas import tpu as pltpu
from jax.experimental.pallas import tpu_sc as plsc

NUM_CORES = 2
NUM_SUBCORES = 16
NW = NUM_CORES * NUM_SUBCORES
LANES = 16
DIM = 64
TOTAL = 2 * 16384
B_PER_W = TOTAL // NW
IDX_ROW = 128
NIDX = B_PER_W // IDX_ROW
CHUNK = 256
NCHUNK = B_PER_W // CHUNK
GPC = CHUNK // IDX_ROW
BLOCKS = CHUNK // LANES


def _sqrt_f32(x):
    i = plsc.bitcast(x, jnp.int32)
    y = plsc.bitcast(jnp.int32(0x5F3759DF) - (i >> 1), jnp.float32)
    for _ in range(3):
        y = y * (1.5 - 0.5 * x * y * y)
    return x * y


def _transe_body(ent_hbm, rel_hbm, h_hbm, r_hbm, t_hbm, out_hbm,
                 hidx, ridx, tidx,
                 hrows0, hrows1, rrows0, rrows1, trows0, trows1,
                 sums, sem):
    wid = lax.axis_index("s") * NUM_CORES + lax.axis_index("c")

    ih = pltpu.async_copy(h_hbm.at[wid], hidx, sem)
    ir = pltpu.async_copy(r_hbm.at[wid], ridx, sem)
    it = pltpu.async_copy(t_hbm.at[wid], tidx, sem)
    ih.wait()
    ir.wait()
    it.wait()

    hbufs = (hrows0, hrows1)
    rbufs = (rrows0, rrows1)
    tbufs = (trows0, trows1)

    def fire(c):
        hb, rb, tb = hbufs[c % 2], rbufs[c % 2], tbufs[c % 2]
        cps = []
        for g in range(GPC):
            j = c * GPC + g
            dst = pl.ds(g * IDX_ROW, IDX_ROW)
            cps.append(pltpu.async_copy(ent_hbm.at[hidx.at[j]], hb.at[dst], sem))
            cps.append(pltpu.async_copy(rel_hbm.at[ridx.at[j]], rb.at[dst], sem))
            cps.append(pltpu.async_copy(ent_hbm.at[tidx.at[j]], tb.at[dst], sem))
        return cps

    lane_iota = lax.iota(jnp.int32, LANES)

    def compute(c):
        hb, rb, tb = hbufs[c % 2], rbufs[c % 2], tbufs[c % 2]

        def blk_body(b, carry):
            rowv = b * LANES + lane_iota
            acc = jnp.zeros((LANES,), jnp.float32)
            for j in range(DIM):
                cj = jnp.full((LANES,), j, jnp.int32)
                hv = plsc.load_gather(hb, [rowv, cj])
                rv = plsc.load_gather(rb, [rowv, cj])
                tv = plsc.load_gather(tb, [rowv, cj])
                d = hv + rv - tv
                acc = acc + d * d
            sums[pl.ds(c * CHUNK + b * LANES, LANES)] = _sqrt_f32(acc)
            return carry

        lax.fori_loop(0, BLOCKS, blk_body, 0)

    inflight = fire(0)
    for c in range(NCHUNK):
        nxt = fire(c + 1) if c + 1 < NCHUNK else []
        for cp in inflight:
            cp.wait()
        compute(c)
        inflight = nxt

    pltpu.sync_copy(sums, out_hbm.at[pl.ds(wid * B_PER_W, B_PER_W)])


@jax.jit
def kernel(entity_embeddings, relation_embeddings,
           positive_head_batch, positive_relation_batch, positive_tail_batch,
           negative_head_batch, negative_relation_batch, negative_tail_batch):
    heads = jnp.concatenate([positive_head_batch, negative_head_batch])
    rels = jnp.concatenate([positive_relation_batch, negative_relation_batch])
    tails = jnp.concatenate([positive_tail_batch, negative_tail_batch])
    heads = heads.astype(jnp.int32).reshape(NW, NIDX, IDX_ROW)
    rels = rels.astype(jnp.int32).reshape(NW, NIDX, IDX_ROW)
    tails = tails.astype(jnp.int32).reshape(NW, NIDX, IDX_ROW)

    k = pl.kernel(
        _transe_body,
        out_type=jax.ShapeDtypeStruct((TOTAL,), jnp.float32),
        mesh=plsc.VectorSubcoreMesh(core_axis_name="c", subcore_axis_name="s"),
        scratch_types=[
            pltpu.VMEM((NIDX, IDX_ROW), jnp.int32),
            pltpu.VMEM((NIDX, IDX_ROW), jnp.int32),
            pltpu.VMEM((NIDX, IDX_ROW), jnp.int32),
            pltpu.VMEM((CHUNK, DIM), jnp.float32),
            pltpu.VMEM((CHUNK, DIM), jnp.float32),
            pltpu.VMEM((CHUNK, DIM), jnp.float32),
            pltpu.VMEM((CHUNK, DIM), jnp.float32),
            pltpu.VMEM((CHUNK, DIM), jnp.float32),
            pltpu.VMEM((CHUNK, DIM), jnp.float32),
            pltpu.VMEM((B_PER_W,), jnp.float32),
            pltpu.SemaphoreType.DMA,
        ],
        compiler_params=pltpu.CompilerParams(
            needs_layout_passes=False, use_tc_tiling_on_sc=False),
        name="transe_sc",
    )
    losses = k(entity_embeddings, relation_embeddings, heads, rels, tails)
    return losses.reshape(2, 16384)
